# baseline (speedup 1.0000x reference)
import functools

import jax
import jax.numpy as jnp
from jax.experimental import pallas as pl
from jax.experimental.pallas import tpu as pltpu
from jax.experimental.pallas import tpu_sc as plsc

_B, _C, _L, _V = 16, 32, 1024, 8192
_SEG = (1, 4, 16, 64, 256, 1024)
_BETA = 0.25
_PI = (0, 0, 1, 2, 3, 3)
_VC = 256


def _quant_body(q_ref, cbT_ref, idx_ref, *, n_rows):
    nchunks = _V // _VC
    q = q_ref[...]

    def p1(c, m):
        sc = jnp.dot(q, cbT_ref[:, pl.ds(c * _VC, _VC)],
                     preferred_element_type=jnp.float32)
        return jnp.maximum(m, jnp.max(sc, axis=1, keepdims=True))

    m = jax.lax.fori_loop(
        0, nchunks, p1, jnp.full((n_rows, 1), -jnp.inf, jnp.float32))

    def p2(c, idx):
        sc = jnp.dot(q, cbT_ref[:, pl.ds(c * _VC, _VC)],
                     preferred_element_type=jnp.float32)
        lane = jax.lax.broadcasted_iota(jnp.int32, (n_rows, _VC), 1) + c * _VC
        targ = jnp.min(jnp.where(sc >= m, lane, _V), axis=1, keepdims=True)
        return jnp.minimum(idx, targ)

    idx_ref[...] = jax.lax.fori_loop(
        0, nchunks, p2, jnp.full((n_rows, 1), _V, jnp.int32))


def _quantize_idx(q, cbT):
    n = q.shape[0]
    rc = min(n, 2048)
    return pl.pallas_call(
        functools.partial(_quant_body, n_rows=rc),
        grid=(n // rc,),
        in_specs=[
            pl.BlockSpec((rc, _C), lambda i: (i, 0)),
            pl.BlockSpec((_C, _V), lambda i: (0, 0)),
        ],
        out_specs=pl.BlockSpec((rc, 1), lambda i: (i, 0)),
        out_shape=jax.ShapeDtypeStruct((n, 1), jnp.int32),
    )(q, cbT)


def _sc_gather_body(emb_hbm, idx_hbm, out_hbm, idx_v, rows_v, sem,
                    *, bpw, nw, chunk):
    wid = jax.lax.axis_index("s") * 2 + jax.lax.axis_index("c")

    @pl.when(wid < nw)
    def _():
        base = wid * bpw
        pltpu.sync_copy(idx_hbm.at[pl.ds(base, bpw)], idx_v)
        copies = []
        for k in range(bpw // chunk):
            copies.append(pltpu.async_copy(
                emb_hbm.at[idx_v.at[pl.ds(k * chunk, chunk)]],
                rows_v.at[pl.ds(k * chunk, chunk)], sem))
        for cp in copies:
            cp.wait()
        pltpu.sync_copy(rows_v, out_hbm.at[pl.ds(base, bpw)])


def _sc_gather(emb_pad, idx_flat):
    n = idx_flat.shape[0]
    nw = min(32, n // 8)
    bpw = n // nw
    chunk = min(128, bpw)
    mesh = plsc.VectorSubcoreMesh(core_axis_name="c", subcore_axis_name="s")
    f = pl.kernel(
        functools.partial(_sc_gather_body, bpw=bpw, nw=nw, chunk=chunk),
        mesh=mesh,
        out_type=jax.ShapeDtypeStruct((n, 128), jnp.float32),
        scratch_types=[
            pltpu.VMEM((bpw,), jnp.int32),
            pltpu.VMEM((bpw, 128), jnp.float32),
            pltpu.SemaphoreType.DMA,
        ],
    )
    return f(emb_pad, idx_flat)


def kernel(f_BCl, emb_W, phi_W, phi_b):
    f = f_BCl.transpose(0, 2, 1)
    norms = jnp.maximum(jnp.linalg.norm(emb_W, axis=1, keepdims=True), 1e-12)
    cbT = (emb_W / norms).T
    emb_pad = jnp.pad(emb_W, ((0, 0), (0, 128 - _C)))

    m_prev = phi_W[:, :, :, 0].transpose(0, 2, 1)
    m_cur = phi_W[:, :, :, 1].transpose(0, 2, 1)
    m_next = phi_W[:, :, :, 2].transpose(0, 2, 1)

    f_rest = f
    f_hat = jnp.zeros_like(f)
    loss = jnp.float32(0.0)
    zrow = jnp.zeros((_B, 1, _C), jnp.float32)
    for si, s in enumerate(_SEG):
        w = _L // s
        pooled = f_rest.reshape(_B, s, w, _C).mean(axis=2)
        q = pooled.reshape(_B * s, _C)
        q = q / jnp.maximum(jnp.linalg.norm(q, axis=1, keepdims=True), 1e-12)
        idx = _quantize_idx(q, cbT).reshape(_B * s)
        hseg = _sc_gather(emb_pad, idx)[:, :_C].reshape(_B, s, _C)
        h = jnp.broadcast_to(hseg[:, :, None, :], (_B, s, w, _C))
        h = h.reshape(_B, _L, _C)
        pi = _PI[si]
        hp = jnp.concatenate([zrow, h[:, :-1]], axis=1)
        hn = jnp.concatenate([h[:, 1:], zrow], axis=1)
        conv = (hp @ m_prev[pi] + h @ m_cur[pi] + hn @ m_next[pi]
                + phi_b[pi][None, None])
        mixed = 0.5 * h + 0.5 * conv
        f_hat = f_hat + mixed
        f_rest = f_rest - mixed
        loss = loss + jnp.mean(f_rest ** 2)
    loss = (1.0 + _BETA) * loss / len(_SEG)
    return f_hat.transpose(0, 2, 1), loss


# speedup vs baseline: 1.0604x; 1.0604x over previous
import functools

import jax
import jax.numpy as jnp
from jax.experimental import pallas as pl
from jax.experimental.pallas import tpu as pltpu
from jax.experimental.pallas import tpu_sc as plsc

_B, _C, _L, _V = 16, 32, 1024, 8192
_N = _B * _L
_SEG = (1, 4, 16, 64, 256, 1024)
_BETA = 0.25
_PI = (0, 0, 1, 2, 3, 3)
_VC = 256
_SLAB = 2048
_RCQ = 2048


def _quant_rows(q, cbT_ref, n_rows):
    def step(c, carry):
        m, idx = carry
        sc = jnp.dot(q, cbT_ref[:, pl.ds(c * _VC, _VC)],
                     preferred_element_type=jnp.float32)
        tmax = jnp.max(sc, axis=1, keepdims=True)
        lane = jax.lax.broadcasted_iota(jnp.int32, (n_rows, _VC), 1) + c * _VC
        targ = jnp.min(jnp.where(sc >= tmax, lane, _V), axis=1, keepdims=True)
        better = tmax > m
        return jnp.maximum(m, tmax), jnp.where(better, targ, idx)

    m0 = jnp.full((n_rows, 1), -jnp.inf, jnp.float32)
    i0 = jnp.zeros((n_rows, 1), jnp.int32)
    _, idx = jax.lax.fori_loop(0, _V // _VC, step, (m0, i0))
    return idx


def _update_slab(frest_ref, hseg_ref, mats_ref, bias_ref, frest_out,
                 a, w_prev, sprev):
    np_rows = 16 * sprev
    ja, jb = a // w_prev, (a + _SLAB) // w_prev
    ns = jb - ja
    hs = hseg_ref[ja:jb, :_C]
    z1 = jnp.zeros((1, _C), jnp.float32)
    prev_seg = hseg_ref[ja - 1:ja, :_C] if ja > 0 else z1
    next_seg = hseg_ref[jb:jb + 1, :_C] if jb < np_rows else z1
    hsp = jnp.concatenate([prev_seg, hs[:-1]], axis=0)
    hsn = jnp.concatenate([hs[1:], next_seg], axis=0)
    j = jax.lax.broadcasted_iota(jnp.int32, (ns, 1), 0) + ja
    hsp = jnp.where((j & (sprev - 1)) == 0, 0.0, hsp)
    hsn = jnp.where((j & (sprev - 1)) == sprev - 1, 0.0, hsn)

    def up(x):
        return jnp.broadcast_to(x[:, None, :], (ns, w_prev, _C)).reshape(
            ns * w_prev, _C)

    h = up(hs)
    if w_prev == 1:
        hp, hn = hsp, hsn
    else:
        l = jax.lax.broadcasted_iota(jnp.int32, (_SLAB, 1), 0)
        hp = jnp.where((l & (w_prev - 1)) == 0, up(hsp), h)
        hn = jnp.where((l & (w_prev - 1)) == w_prev - 1, up(hsn), h)
    conv = (jnp.dot(hp, mats_ref[0:_C, :], preferred_element_type=jnp.float32)
            + jnp.dot(h, mats_ref[_C:2 * _C, :],
                      preferred_element_type=jnp.float32)
            + jnp.dot(hn, mats_ref[2 * _C:3 * _C, :],
                      preferred_element_type=jnp.float32)
            + bias_ref[0:1, :])
    mixed = 0.5 * h + 0.5 * conv
    fr = frest_ref[a:a + _SLAB, :] - mixed
    frest_out[a:a + _SLAB, :] = fr
    return jnp.sum(fr * fr, keepdims=True)


def _quant_stage(src_ref, cbT_ref, idx_ref, w, n_rows):
    rcq = min(n_rows, _RCQ)
    for sl in range(n_rows // rcq):
        rows = src_ref[sl * rcq * w:(sl + 1) * rcq * w, :]
        pooled = jnp.mean(rows.reshape(rcq, w, _C), axis=1) if w > 1 else rows
        nrm = jnp.sqrt(jnp.sum(pooled * pooled, axis=1, keepdims=True))
        q = pooled / jnp.maximum(nrm, 1e-12)
        idx_ref[sl * rcq:(sl + 1) * rcq, :] = _quant_rows(q, cbT_ref, rcq)


def _first_body(frest_ref, cbT_ref, idx_ref, *, w, n_rows):
    _quant_stage(frest_ref, cbT_ref, idx_ref, w, n_rows)


def _scale_body(frest_ref, hseg_ref, mats_ref, bias_ref, cbT_ref,
                frest_out, idx_ref, loss_ref, *, w_prev, sprev, w, n_rows):
    acc = jnp.zeros((1, 1), jnp.float32)
    for sl in range(_N // _SLAB):
        acc = acc + _update_slab(frest_ref, hseg_ref, mats_ref, bias_ref,
                                 frest_out, sl * _SLAB, w_prev, sprev)
    loss_ref[...] = acc
    _quant_stage(frest_out, cbT_ref, idx_ref, w, n_rows)


def _final_body(frest_ref, hseg_ref, mats_ref, bias_ref,
                frest_out, loss_ref, *, w_prev, sprev):
    acc = jnp.zeros((1, 1), jnp.float32)
    for sl in range(_N // _SLAB):
        acc = acc + _update_slab(frest_ref, hseg_ref, mats_ref, bias_ref,
                                 frest_out, sl * _SLAB, w_prev, sprev)
    loss_ref[...] = acc


def _first_call(f, cbT):
    n = 16 * _SEG[0]
    return pl.pallas_call(
        functools.partial(_first_body, w=_L // _SEG[0], n_rows=n),
        out_shape=jax.ShapeDtypeStruct((n, 1), jnp.int32),
    )(f, cbT)


def _scale_call(si, frest, hseg, mats, bias, cbT):
    n = 16 * _SEG[si]
    return pl.pallas_call(
        functools.partial(_scale_body, w_prev=_L // _SEG[si - 1],
                          sprev=_SEG[si - 1], w=_L // _SEG[si], n_rows=n),
        out_shape=(jax.ShapeDtypeStruct((_N, _C), jnp.float32),
                   jax.ShapeDtypeStruct((n, 1), jnp.int32),
                   jax.ShapeDtypeStruct((1, 1), jnp.float32)),
        input_output_aliases={0: 0},
    )(frest, hseg, mats, bias, cbT)


def _final_call(frest, hseg, mats, bias):
    return pl.pallas_call(
        functools.partial(_final_body, w_prev=_L // _SEG[-1],
                          sprev=_SEG[-1]),
        out_shape=(jax.ShapeDtypeStruct((_N, _C), jnp.float32),
                   jax.ShapeDtypeStruct((1, 1), jnp.float32)),
        input_output_aliases={0: 0},
    )(frest, hseg, mats, bias)


def _sc_gather_body(emb_hbm, idx_hbm, out_hbm, idx_v, rows_v, sem,
                    *, bpw, nw, chunk):
    wid = jax.lax.axis_index("s") * 2 + jax.lax.axis_index("c")

    @pl.when(wid < nw)
    def _():
        base = wid * bpw
        pltpu.sync_copy(idx_hbm.at[pl.ds(base, bpw)], idx_v)
        copies = []
        for k in range(bpw // chunk):
            copies.append(pltpu.async_copy(
                emb_hbm.at[idx_v.at[pl.ds(k * chunk, chunk)]],
                rows_v.at[pl.ds(k * chunk, chunk)], sem))
        for cp in copies:
            cp.wait()
        pltpu.sync_copy(rows_v, out_hbm.at[pl.ds(base, bpw)])


def _sc_gather(emb_pad, idx_flat):
    n = idx_flat.shape[0]
    nw = min(32, n // 8)
    bpw = n // nw
    chunk = min(128, bpw)
    mesh = plsc.VectorSubcoreMesh(core_axis_name="c", subcore_axis_name="s")
    f = pl.kernel(
        functools.partial(_sc_gather_body, bpw=bpw, nw=nw, chunk=chunk),
        mesh=mesh,
        out_type=jax.ShapeDtypeStruct((n, 128), jnp.float32),
        scratch_types=[
            pltpu.VMEM((bpw,), jnp.int32),
            pltpu.VMEM((bpw, 128), jnp.float32),
            pltpu.SemaphoreType.DMA,
        ],
    )
    return f(emb_pad, idx_flat)


def kernel(f_BCl, emb_W, phi_W, phi_b):
    f = f_BCl.transpose(0, 2, 1).reshape(_N, _C)
    norms = jnp.maximum(jnp.linalg.norm(emb_W, axis=1, keepdims=True), 1e-12)
    cbT = (emb_W / norms).T
    emb_pad = jnp.pad(emb_W, ((0, 0), (0, 128 - _C)))

    mats = jnp.concatenate([
        phi_W[:, :, :, 0].transpose(0, 2, 1),
        phi_W[:, :, :, 1].transpose(0, 2, 1),
        phi_W[:, :, :, 2].transpose(0, 2, 1),
    ], axis=1)
    bias = phi_b[:, None, :]

    idx = _first_call(f, cbT).reshape(16 * _SEG[0])
    hseg = _sc_gather(emb_pad, idx)
    frest = f
    losses = []
    for si in range(1, len(_SEG)):
        pi = _PI[si - 1]
        frest, idx, ls = _scale_call(si, frest, hseg, mats[pi], bias[pi], cbT)
        losses.append(ls[0, 0])
        hseg = _sc_gather(emb_pad, idx.reshape(16 * _SEG[si]))
    pi = _PI[-1]
    frest, ls = _final_call(frest, hseg, mats[pi], bias[pi])
    losses.append(ls[0, 0])

    loss = jnp.float32(0.0)
    for ls in losses:
        loss = loss + ls / jnp.float32(_N * _C)
    loss = (1.0 + _BETA) * loss / len(_SEG)
    f_hat = (f - frest).reshape(_B, _L, _C).transpose(0, 2, 1)
    return f_hat, loss


# speedup vs baseline: 1.5896x; 1.4990x over previous
import functools

import jax
import jax.numpy as jnp
from jax.experimental import pallas as pl
from jax.experimental.pallas import tpu as pltpu
from jax.experimental.pallas import tpu_sc as plsc

_B, _C, _L, _V = 16, 32, 1024, 8192
_SEG = (1, 4, 16, 64, 256, 1024)
_BETA = 0.25
_PI = (0, 0, 1, 2, 3, 3)
_VC = 512


def _quant_body(q_ref, cbT_ref, idx_ref, *, n_rows):
    nchunks = _V // _VC
    q = q_ref[...]

    def step(c, carry):
        m, idx = carry
        sc = jnp.dot(q, cbT_ref[:, pl.ds(c * _VC, _VC)],
                     preferred_element_type=jnp.float32)
        tmax = jnp.max(sc, axis=1, keepdims=True)
        lane = jax.lax.broadcasted_iota(jnp.int32, (n_rows, _VC), 1) + c * _VC
        targ = jnp.min(jnp.where(sc >= tmax, lane, _V), axis=1, keepdims=True)
        better = tmax > m
        return jnp.maximum(m, tmax), jnp.where(better, targ, idx)

    m0 = jnp.full((n_rows, 1), -jnp.inf, jnp.float32)
    i0 = jnp.zeros((n_rows, 1), jnp.int32)
    _, idx = jax.lax.fori_loop(0, nchunks, step, (m0, i0))
    idx_ref[...] = idx


def _quantize_idx(q, cbT):
    n = q.shape[0]
    rc = min(n, 2048)
    return pl.pallas_call(
        functools.partial(_quant_body, n_rows=rc),
        grid=(n // rc,),
        in_specs=[
            pl.BlockSpec((rc, _C), lambda i: (i, 0)),
            pl.BlockSpec((_C, _V), lambda i: (0, 0)),
        ],
        out_specs=pl.BlockSpec((rc, 1), lambda i: (i, 0)),
        out_shape=jax.ShapeDtypeStruct((n, 1), jnp.int32),
    )(q, cbT)


def _sc_gather_body(emb_hbm, idx_hbm, out_hbm, idx_v, rows_v, sem,
                    *, bpw, nw, chunk):
    wid = jax.lax.axis_index("s") * 2 + jax.lax.axis_index("c")

    @pl.when(wid < nw)
    def _():
        base = wid * bpw
        pltpu.sync_copy(idx_hbm.at[pl.ds(base, bpw)], idx_v)
        copies = []
        for k in range(bpw // chunk):
            copies.append(pltpu.async_copy(
                emb_hbm.at[idx_v.at[pl.ds(k * chunk, chunk)]],
                rows_v.at[pl.ds(k * chunk, chunk)], sem))
        for cp in copies:
            cp.wait()
        pltpu.sync_copy(rows_v, out_hbm.at[pl.ds(base, bpw)])


def _sc_gather(emb_pad, idx_flat):
    n = idx_flat.shape[0]
    nw = min(32, n // 8)
    bpw = n // nw
    chunk = min(128, bpw)
    mesh = plsc.VectorSubcoreMesh(core_axis_name="c", subcore_axis_name="s")
    f = pl.kernel(
        functools.partial(_sc_gather_body, bpw=bpw, nw=nw, chunk=chunk),
        mesh=mesh,
        out_type=jax.ShapeDtypeStruct((n, 128), jnp.float32),
        scratch_types=[
            pltpu.VMEM((bpw,), jnp.int32),
            pltpu.VMEM((bpw, 128), jnp.float32),
            pltpu.SemaphoreType.DMA,
        ],
    )
    return f(emb_pad, idx_flat)


def kernel(f_BCl, emb_W, phi_W, phi_b):
    f = f_BCl.transpose(0, 2, 1)
    norms = jnp.maximum(jnp.linalg.norm(emb_W, axis=1, keepdims=True), 1e-12)
    cbT = (emb_W / norms).T
    emb_pad = jnp.pad(emb_W, ((0, 0), (0, 128 - _C)))

    m_prev = phi_W[:, :, :, 0].transpose(0, 2, 1)
    m_cur = phi_W[:, :, :, 1].transpose(0, 2, 1)
    m_next = phi_W[:, :, :, 2].transpose(0, 2, 1)

    f_rest = f
    f_hat = jnp.zeros_like(f)
    loss = jnp.float32(0.0)
    zrow = jnp.zeros((_B, 1, _C), jnp.float32)
    for si, s in enumerate(_SEG):
        w = _L // s
        pooled = f_rest.reshape(_B, s, w, _C).mean(axis=2)
        q = pooled.reshape(_B * s, _C)
        q = q / jnp.maximum(jnp.linalg.norm(q, axis=1, keepdims=True), 1e-12)
        idx = _quantize_idx(q, cbT).reshape(_B * s)
        hseg = _sc_gather(emb_pad, idx)[:, :_C].reshape(_B, s, _C)
        h = jnp.broadcast_to(hseg[:, :, None, :], (_B, s, w, _C))
        h = h.reshape(_B, _L, _C)
        pi = _PI[si]
        hp = jnp.concatenate([zrow, h[:, :-1]], axis=1)
        hn = jnp.concatenate([h[:, 1:], zrow], axis=1)
        conv = (hp @ m_prev[pi] + h @ m_cur[pi] + hn @ m_next[pi]
                + phi_b[pi][None, None])
        mixed = 0.5 * h + 0.5 * conv
        f_hat = f_hat + mixed
        f_rest = f_rest - mixed
        loss = loss + jnp.mean(f_rest ** 2)
    loss = (1.0 + _BETA) * loss / len(_SEG)
    return f_hat.transpose(0, 2, 1), loss


# speedup vs baseline: 1.7876x; 1.1246x over previous
import functools

import jax
import jax.numpy as jnp
from jax.experimental import pallas as pl
from jax.experimental.pallas import tpu as pltpu
from jax.experimental.pallas import tpu_sc as plsc

_B, _C, _L, _V = 16, 32, 1024, 8192
_SEG = (1, 4, 16, 64, 256, 1024)
_BETA = 0.25
_PI = (0, 0, 1, 2, 3, 3)
_VC = 1024


def _quant_body(q_ref, cbT_ref, idx_ref, *, n_rows):
    nchunks = _V // _VC
    q = q_ref[...]

    def step(c, carry):
        m, idx = carry
        sc = jnp.dot(q, cbT_ref[:, pl.ds(c * _VC, _VC)],
                     preferred_element_type=jnp.float32)
        tmax = jnp.max(sc, axis=1, keepdims=True)
        lane = jax.lax.broadcasted_iota(jnp.int32, (n_rows, _VC), 1) + c * _VC
        targ = jnp.min(jnp.where(sc >= tmax, lane, _V), axis=1, keepdims=True)
        better = tmax > m
        return jnp.maximum(m, tmax), jnp.where(better, targ, idx)

    m0 = jnp.full((n_rows, 1), -jnp.inf, jnp.float32)
    i0 = jnp.zeros((n_rows, 1), jnp.int32)
    _, idx = jax.lax.fori_loop(0, nchunks, step, (m0, i0))
    idx_ref[...] = idx


def _quantize_idx(q, cbT):
    n = q.shape[0]
    rc = min(n, 2048)
    return pl.pallas_call(
        functools.partial(_quant_body, n_rows=rc),
        grid=(n // rc,),
        in_specs=[
            pl.BlockSpec((rc, _C), lambda i: (i, 0)),
            pl.BlockSpec((_C, _V), lambda i: (0, 0)),
        ],
        out_specs=pl.BlockSpec((rc, 1), lambda i: (i, 0)),
        out_shape=jax.ShapeDtypeStruct((n, 1), jnp.int32),
    )(q, cbT)


def _sc_gather_body(emb_hbm, idx_hbm, out_hbm, idx_v, rows_v, sem,
                    *, bpw, nw, chunk):
    wid = jax.lax.axis_index("s") * 2 + jax.lax.axis_index("c")

    @pl.when(wid < nw)
    def _():
        base = wid * bpw
        pltpu.sync_copy(idx_hbm.at[pl.ds(base, bpw)], idx_v)
        copies = []
        for k in range(bpw // chunk):
            copies.append(pltpu.async_copy(
                emb_hbm.at[idx_v.at[pl.ds(k * chunk, chunk)]],
                rows_v.at[pl.ds(k * chunk, chunk)], sem))
        for cp in copies:
            cp.wait()
        pltpu.sync_copy(rows_v, out_hbm.at[pl.ds(base, bpw)])


def _sc_gather(emb_pad, idx_flat):
    n = idx_flat.shape[0]
    nw = min(32, n // 8)
    bpw = n // nw
    chunk = min(128, bpw)
    mesh = plsc.VectorSubcoreMesh(core_axis_name="c", subcore_axis_name="s")
    f = pl.kernel(
        functools.partial(_sc_gather_body, bpw=bpw, nw=nw, chunk=chunk),
        mesh=mesh,
        out_type=jax.ShapeDtypeStruct((n, 128), jnp.float32),
        scratch_types=[
            pltpu.VMEM((bpw,), jnp.int32),
            pltpu.VMEM((bpw, 128), jnp.float32),
            pltpu.SemaphoreType.DMA,
        ],
    )
    return f(emb_pad, idx_flat)


def kernel(f_BCl, emb_W, phi_W, phi_b):
    f = f_BCl.transpose(0, 2, 1)
    norms = jnp.maximum(jnp.linalg.norm(emb_W, axis=1, keepdims=True), 1e-12)
    cbT = (emb_W / norms).T
    emb_pad = jnp.pad(emb_W, ((0, 0), (0, 128 - _C)))

    m_prev = phi_W[:, :, :, 0].transpose(0, 2, 1)
    m_cur = phi_W[:, :, :, 1].transpose(0, 2, 1)
    m_next = phi_W[:, :, :, 2].transpose(0, 2, 1)

    f_rest = f
    f_hat = jnp.zeros_like(f)
    loss = jnp.float32(0.0)
    zrow = jnp.zeros((_B, 1, _C), jnp.float32)
    for si, s in enumerate(_SEG):
        w = _L // s
        pooled = f_rest.reshape(_B, s, w, _C).mean(axis=2)
        q = pooled.reshape(_B * s, _C)
        q = q / jnp.maximum(jnp.linalg.norm(q, axis=1, keepdims=True), 1e-12)
        idx = _quantize_idx(q, cbT).reshape(_B * s)
        hseg = _sc_gather(emb_pad, idx)[:, :_C].reshape(_B, s, _C)
        h = jnp.broadcast_to(hseg[:, :, None, :], (_B, s, w, _C))
        h = h.reshape(_B, _L, _C)
        pi = _PI[si]
        hp = jnp.concatenate([zrow, h[:, :-1]], axis=1)
        hn = jnp.concatenate([h[:, 1:], zrow], axis=1)
        conv = (hp @ m_prev[pi] + h @ m_cur[pi] + hn @ m_next[pi]
                + phi_b[pi][None, None])
        mixed = 0.5 * h + 0.5 * conv
        f_hat = f_hat + mixed
        f_rest = f_rest - mixed
        loss = loss + jnp.mean(f_rest ** 2)
    loss = (1.0 + _BETA) * loss / len(_SEG)
    return f_hat.transpose(0, 2, 1), loss


# speedup vs baseline: 1.9237x; 1.0761x over previous
import functools

import jax
import jax.numpy as jnp
from jax.experimental import pallas as pl
from jax.experimental.pallas import tpu as pltpu
from jax.experimental.pallas import tpu_sc as plsc

_B, _C, _L, _V = 16, 32, 1024, 8192
_SEG = (1, 4, 16, 64, 256, 1024)
_BETA = 0.25
_PI = (0, 0, 1, 2, 3, 3)
_VC = 2048


def _quant_body(q_ref, cbT_ref, idx_ref, *, n_rows):
    nchunks = _V // _VC
    q = q_ref[...]

    def step(c, carry):
        m, idx = carry
        sc = jnp.dot(q, cbT_ref[:, pl.ds(c * _VC, _VC)],
                     preferred_element_type=jnp.float32)
        tmax = jnp.max(sc, axis=1, keepdims=True)
        lane = jax.lax.broadcasted_iota(jnp.int32, (n_rows, _VC), 1) + c * _VC
        targ = jnp.min(jnp.where(sc >= tmax, lane, _V), axis=1, keepdims=True)
        better = tmax > m
        return jnp.maximum(m, tmax), jnp.where(better, targ, idx)

    m0 = jnp.full((n_rows, 1), -jnp.inf, jnp.float32)
    i0 = jnp.zeros((n_rows, 1), jnp.int32)
    _, idx = jax.lax.fori_loop(0, nchunks, step, (m0, i0))
    idx_ref[...] = idx


def _quantize_idx(q, cbT):
    n = q.shape[0]
    rc = min(n, 2048)
    return pl.pallas_call(
        functools.partial(_quant_body, n_rows=rc),
        grid=(n // rc,),
        in_specs=[
            pl.BlockSpec((rc, _C), lambda i: (i, 0)),
            pl.BlockSpec((_C, _V), lambda i: (0, 0)),
        ],
        out_specs=pl.BlockSpec((rc, 1), lambda i: (i, 0)),
        out_shape=jax.ShapeDtypeStruct((n, 1), jnp.int32),
    )(q, cbT)


def _sc_gather_body(emb_hbm, idx_hbm, out_hbm, idx_v, rows_v, sem,
                    *, bpw, nw, chunk):
    wid = jax.lax.axis_index("s") * 2 + jax.lax.axis_index("c")

    @pl.when(wid < nw)
    def _():
        base = wid * bpw
        pltpu.sync_copy(idx_hbm.at[pl.ds(base, bpw)], idx_v)
        copies = []
        for k in range(bpw // chunk):
            copies.append(pltpu.async_copy(
                emb_hbm.at[idx_v.at[pl.ds(k * chunk, chunk)]],
                rows_v.at[pl.ds(k * chunk, chunk)], sem))
        for cp in copies:
            cp.wait()
        pltpu.sync_copy(rows_v, out_hbm.at[pl.ds(base, bpw)])


def _sc_gather(emb_pad, idx_flat):
    n = idx_flat.shape[0]
    nw = min(32, n // 8)
    bpw = n // nw
    chunk = min(128, bpw)
    mesh = plsc.VectorSubcoreMesh(core_axis_name="c", subcore_axis_name="s")
    f = pl.kernel(
        functools.partial(_sc_gather_body, bpw=bpw, nw=nw, chunk=chunk),
        mesh=mesh,
        out_type=jax.ShapeDtypeStruct((n, 128), jnp.float32),
        scratch_types=[
            pltpu.VMEM((bpw,), jnp.int32),
            pltpu.VMEM((bpw, 128), jnp.float32),
            pltpu.SemaphoreType.DMA,
        ],
    )
    return f(emb_pad, idx_flat)


def kernel(f_BCl, emb_W, phi_W, phi_b):
    f = f_BCl.transpose(0, 2, 1)
    norms = jnp.maximum(jnp.linalg.norm(emb_W, axis=1, keepdims=True), 1e-12)
    cbT = (emb_W / norms).T
    emb_pad = jnp.pad(emb_W, ((0, 0), (0, 128 - _C)))

    m_prev = phi_W[:, :, :, 0].transpose(0, 2, 1)
    m_cur = phi_W[:, :, :, 1].transpose(0, 2, 1)
    m_next = phi_W[:, :, :, 2].transpose(0, 2, 1)

    f_rest = f
    f_hat = jnp.zeros_like(f)
    loss = jnp.float32(0.0)
    zrow = jnp.zeros((_B, 1, _C), jnp.float32)
    for si, s in enumerate(_SEG):
        w = _L // s
        pooled = f_rest.reshape(_B, s, w, _C).mean(axis=2)
        q = pooled.reshape(_B * s, _C)
        q = q / jnp.maximum(jnp.linalg.norm(q, axis=1, keepdims=True), 1e-12)
        idx = _quantize_idx(q, cbT).reshape(_B * s)
        hseg = _sc_gather(emb_pad, idx)[:, :_C].reshape(_B, s, _C)
        h = jnp.broadcast_to(hseg[:, :, None, :], (_B, s, w, _C))
        h = h.reshape(_B, _L, _C)
        pi = _PI[si]
        hp = jnp.concatenate([zrow, h[:, :-1]], axis=1)
        hn = jnp.concatenate([h[:, 1:], zrow], axis=1)
        conv = (hp @ m_prev[pi] + h @ m_cur[pi] + hn @ m_next[pi]
                + phi_b[pi][None, None])
        mixed = 0.5 * h + 0.5 * conv
        f_hat = f_hat + mixed
        f_rest = f_rest - mixed
        loss = loss + jnp.mean(f_rest ** 2)
    loss = (1.0 + _BETA) * loss / len(_SEG)
    return f_hat.transpose(0, 2, 1), loss
